# initial kernel scaffold (unmeasured)
import jax
import jax.numpy as jnp
from jax import lax
from jax.experimental import pallas as pl
from jax.experimental.pallas import tpu as pltpu

CH = 64


def kernel(x, dest):
    t, d = x.shape
    maxc = t // CH

    order = jnp.argsort(dest, stable=True)
    xs = x[order]
    c0 = jnp.sum(dest == 0).astype(jnp.int32)
    pad = jnp.zeros((CH, d), x.dtype)
    xs_pad = jnp.concatenate([pad, xs, pad], axis=0)
    cnt = jnp.reshape(c0, (1,))

    def body(cnt_ref, xs_ref, out_ref, send_sems, recv_sems):
        mx = lax.axis_index("x")
        my = lax.axis_index("y")
        mz = lax.axis_index("z")
        partner = (1 - mx, my, mz)

        barrier_sem = pltpu.get_barrier_semaphore()
        pl.semaphore_signal(
            barrier_sem,
            inc=1,
            device_id=partner,
            device_id_type=pl.DeviceIdType.MESH,
        )
        pl.semaphore_wait(barrier_sem, 1)

        c0v = cnt_ref[0]
        is0 = mx == 0
        cs = jnp.where(is0, t - c0v, c0v)
        n = (cs + CH - 1) // CH

        def chunk_copy(k):
            src = jnp.where(is0, CH + c0v + k * CH, CH + c0v - (k + 1) * CH)
            dst = jnp.where(is0, k * CH, t - (k + 1) * CH)
            return pltpu.make_async_remote_copy(
                src_ref=xs_ref.at[pl.ds(src, CH), :],
                dst_ref=out_ref.at[pl.ds(dst, CH), :],
                send_sem=send_sems.at[k],
                recv_sem=recv_sems.at[k],
                device_id=partner,
                device_id_type=pl.DeviceIdType.MESH,
            )

        for k in range(maxc):
            @pl.when(k < n)
            def _(k=k):
                chunk_copy(k).start()

        for k in range(maxc):
            @pl.when(k < n)
            def _(k=k):
                chunk_copy(k).wait_recv()

        rows = lax.broadcasted_iota(jnp.int32, (t, 1), 0)
        own_mask = jnp.where(is0, rows < c0v, rows >= c0v)
        own = xs_ref[pl.ds(CH, t), :]
        out_ref[:, :] = jnp.where(own_mask, own, out_ref[:, :])

        for k in range(maxc):
            @pl.when(k < n)
            def _(k=k):
                chunk_copy(k).wait_send()

    return pl.pallas_call(
        body,
        out_shape=jax.ShapeDtypeStruct((t, d), x.dtype),
        in_specs=[
            pl.BlockSpec(memory_space=pltpu.SMEM),
            pl.BlockSpec(memory_space=pltpu.VMEM),
        ],
        out_specs=pl.BlockSpec(memory_space=pltpu.VMEM),
        scratch_shapes=[
            pltpu.SemaphoreType.DMA((maxc,)),
            pltpu.SemaphoreType.DMA((maxc,)),
        ],
        compiler_params=pltpu.CompilerParams(collective_id=0),
    )(cnt, xs_pad)


# baseline (device time: 28801 ns/iter reference)
import jax
import jax.numpy as jnp
from jax import lax
from jax.experimental import pallas as pl
from jax.experimental.pallas import tpu as pltpu

CH = 64


def kernel(x, dest):
    t, d = x.shape
    maxc = t // CH

    order = jnp.argsort(dest, stable=True)
    xs = x[order]
    c0 = jnp.sum(dest == 0).astype(jnp.int32)
    send_buf = jnp.roll(xs, t - c0, axis=0)
    cnt = jnp.reshape(c0, (1,))

    def body(cnt_ref, xs_ref, send_ref, out_ref, send_sems, recv_sems):
        mx = lax.axis_index("x")
        my = lax.axis_index("y")
        mz = lax.axis_index("z")
        partner = (1 - mx, my, mz)

        barrier_sem = pltpu.get_barrier_semaphore()
        pl.semaphore_signal(
            barrier_sem,
            inc=1,
            device_id=partner,
            device_id_type=pl.DeviceIdType.MESH,
        )
        pl.semaphore_wait(barrier_sem, 1)

        c0v = cnt_ref[0]
        is0 = mx == 0
        cs = jnp.where(is0, t - c0v, c0v)
        n = (cs + CH - 1) // CH

        def chunk_copy(k):
            off = jnp.where(is0, k * CH, t - (k + 1) * CH)
            return pltpu.make_async_remote_copy(
                src_ref=send_ref.at[pl.ds(off, CH), :],
                dst_ref=out_ref.at[pl.ds(off, CH), :],
                send_sem=send_sems.at[k],
                recv_sem=recv_sems.at[k],
                device_id=partner,
                device_id_type=pl.DeviceIdType.MESH,
            )

        for k in range(maxc):
            @pl.when(k < n)
            def _(k=k):
                chunk_copy(k).start()

        for k in range(maxc):
            @pl.when(k < n)
            def _(k=k):
                chunk_copy(k).wait_recv()

        rows = lax.broadcasted_iota(jnp.int32, (t, 1), 0)
        s = 1 - 2 * mx
        own_mask = (s * rows) < (s * c0v + mx)
        out_ref[:, :] = jnp.where(own_mask, xs_ref[:, :], out_ref[:, :])

        for k in range(maxc):
            @pl.when(k < n)
            def _(k=k):
                chunk_copy(k).wait_send()

    return pl.pallas_call(
        body,
        out_shape=jax.ShapeDtypeStruct((t, d), x.dtype),
        in_specs=[
            pl.BlockSpec(memory_space=pltpu.SMEM),
            pl.BlockSpec(memory_space=pltpu.VMEM),
            pl.BlockSpec(memory_space=pltpu.VMEM),
        ],
        out_specs=pl.BlockSpec(memory_space=pltpu.VMEM),
        scratch_shapes=[
            pltpu.SemaphoreType.DMA((maxc,)),
            pltpu.SemaphoreType.DMA((maxc,)),
        ],
        compiler_params=pltpu.CompilerParams(collective_id=0),
    )(cnt, xs, send_buf)
